# Initial kernel scaffold; baseline (speedup 1.0000x reference)
import jax
import jax.numpy as jnp
from jax.experimental import pallas as pl


def kernel(x, pos_table):
    raise NotImplementedError("write your pallas kernel here")


# speedup vs baseline: 2.8513x; 2.8513x over previous
import jax
import jax.numpy as jnp
from jax.experimental import pallas as pl


_BLK_S = 512


def _add_body(x_ref, pos_ref, o_ref):
    o_ref[...] = x_ref[...] + pos_ref[...]


def kernel(x, pos_table):
    batch, seq, d = x.shape
    blk = _BLK_S
    grid = (seq // blk, batch)
    return pl.pallas_call(
        _add_body,
        grid=grid,
        in_specs=[
            pl.BlockSpec((1, blk, d), lambda i, b: (b, i, 0)),
            pl.BlockSpec((blk, d), lambda i, b: (i, 0)),
        ],
        out_specs=pl.BlockSpec((1, blk, d), lambda i, b: (b, i, 0)),
        out_shape=jax.ShapeDtypeStruct((batch, seq, d), x.dtype),
    )(x, pos_table[:seq])


# speedup vs baseline: 3.1820x; 1.1160x over previous
import jax
import jax.numpy as jnp
from jax.experimental import pallas as pl


_BLK_S = 1024


def _add_body(x_ref, pos_ref, o_ref):
    o_ref[...] = x_ref[...] + pos_ref[...]


def kernel(x, pos_table):
    batch, seq, d = x.shape
    blk = _BLK_S
    grid = (seq // blk, batch)
    return pl.pallas_call(
        _add_body,
        grid=grid,
        in_specs=[
            pl.BlockSpec((1, blk, d), lambda i, b: (b, i, 0)),
            pl.BlockSpec((blk, d), lambda i, b: (i, 0)),
        ],
        out_specs=pl.BlockSpec((1, blk, d), lambda i, b: (b, i, 0)),
        out_shape=jax.ShapeDtypeStruct((batch, seq, d), x.dtype),
    )(x, pos_table[:seq])


# speedup vs baseline: 3.3081x; 1.0396x over previous
import jax
import jax.numpy as jnp
from jax.experimental import pallas as pl


_BLK_S = 2048


def _add_body(x_ref, pos_ref, o_ref):
    o_ref[...] = x_ref[...] + pos_ref[...]


def kernel(x, pos_table):
    batch, seq, d = x.shape
    blk = _BLK_S
    grid = (seq // blk, batch)
    return pl.pallas_call(
        _add_body,
        grid=grid,
        in_specs=[
            pl.BlockSpec((1, blk, d), lambda i, b: (b, i, 0)),
            pl.BlockSpec((blk, d), lambda i, b: (i, 0)),
        ],
        out_specs=pl.BlockSpec((1, blk, d), lambda i, b: (b, i, 0)),
        out_shape=jax.ShapeDtypeStruct((batch, seq, d), x.dtype),
    )(x, pos_table[:seq])
